# baseline (speedup 1.0000x reference)
import functools

import jax
import jax.numpy as jnp
from jax import lax
from jax.experimental import pallas as pl
from jax.experimental.pallas import tpu as pltpu
from jax.experimental.pallas import tpu_sc as plsc

BATCH = 16384
DIM = 64
NW = 32
BPW = BATCH // NW
NCHUNK = 4
CHUNK = BPW // NCHUNK


def _sc_dots(u_hbm, ip_hbm, in_hbm, utab_hbm, itab_hbm, x_hbm,
             iu_v, ii_v, ij_v, pu_v, qi_v, qj_v, x_v, sem):
  wid = lax.axis_index("s") * 2 + lax.axis_index("c")
  base = wid * BPW

  pltpu.sync_copy(u_hbm.at[wid], iu_v)
  pltpu.sync_copy(ip_hbm.at[wid], ii_v)
  pltpu.sync_copy(in_hbm.at[wid], ij_v)

  copies = []
  for c in range(NCHUNK):
    dst = pl.ds(c * CHUNK, CHUNK)
    copies.append(pltpu.async_copy(utab_hbm.at[iu_v.at[c]], pu_v.at[dst], sem))
    copies.append(pltpu.async_copy(itab_hbm.at[ii_v.at[c]], qi_v.at[dst], sem))
    copies.append(pltpu.async_copy(itab_hbm.at[ij_v.at[c]], qj_v.at[dst], sem))
  for cp in copies:
    cp.wait()

  lane = lax.iota(jnp.int32, 16)

  def body(g, carry):
    acc16 = jnp.zeros((16,), jnp.float32)
    for r in range(16):
      row = g * 16 + r
      s = jnp.zeros((16,), jnp.float32)
      for c in range(4):
        cs = pl.ds(c * 16, 16)
        s = s + pu_v[row, cs] * (qi_v[row, cs] - qj_v[row, cs])
      acc16 = jnp.where(lane == r, jnp.sum(s), acc16)
    x_v[pl.ds(g * 16, 16)] = acc16
    return carry

  lax.fori_loop(0, BPW // 16, body, 0)

  pltpu.sync_copy(x_v, x_hbm.at[pl.ds(base, BPW)])


@jax.jit
def _sc_stage(u, i_pos, i_neg, user_table, item_table):
  mesh = plsc.VectorSubcoreMesh(core_axis_name="c", subcore_axis_name="s")
  f = pl.kernel(
      _sc_dots,
      out_type=jax.ShapeDtypeStruct((BATCH,), jnp.float32),
      mesh=mesh,
      compiler_params=pltpu.CompilerParams(
          needs_layout_passes=False, use_tc_tiling_on_sc=False),
      scratch_types=[
          pltpu.VMEM((NCHUNK, CHUNK), jnp.int32),
          pltpu.VMEM((NCHUNK, CHUNK), jnp.int32),
          pltpu.VMEM((NCHUNK, CHUNK), jnp.int32),
          pltpu.VMEM((BPW, DIM), jnp.float32),
          pltpu.VMEM((BPW, DIM), jnp.float32),
          pltpu.VMEM((BPW, DIM), jnp.float32),
          pltpu.VMEM((BPW,), jnp.float32),
          pltpu.SemaphoreType.DMA,
      ],
  )
  return f(u, i_pos, i_neg, user_table, item_table)


def _loss_body(x_ref, out_ref):
  x = x_ref[...]
  t = jnp.log(jax.nn.sigmoid(x) + 1e-08)
  out_ref[0, 0] = -jnp.sum(t) * (1.0 / BATCH)


@jax.jit
def _tc_loss(x):
  res = pl.pallas_call(
      _loss_body,
      out_shape=jax.ShapeDtypeStruct((1, 1), jnp.float32),
      out_specs=pl.BlockSpec(memory_space=pltpu.SMEM),
  )(x.reshape(128, 128))
  return res[0, 0]


def kernel(u, i_pos, i_neg, user_table, item_table):
  u = u.astype(jnp.int32).reshape(NW, NCHUNK, CHUNK)
  i_pos = i_pos.astype(jnp.int32).reshape(NW, NCHUNK, CHUNK)
  i_neg = i_neg.astype(jnp.int32).reshape(NW, NCHUNK, CHUNK)
  x = _sc_stage(u, i_pos, i_neg, user_table, item_table)
  return _tc_loss(x)


# speedup vs baseline: 1.5540x; 1.5540x over previous
import functools

import jax
import jax.numpy as jnp
from jax import lax
from jax.experimental import pallas as pl
from jax.experimental.pallas import tpu as pltpu
from jax.experimental.pallas import tpu_sc as plsc

BATCH = 16384
DIM = 64
NW = 32
BPW = BATCH // NW
LAG = 16
CH = 256


def _sc_dots(u_hbm, ip_hbm, in_hbm, utab_hbm, itab_hbm, x_hbm,
             iu_s, ii_s, ij_s, pu_v, qi_v, qj_v, x_v, sem):
  wid = lax.axis_index("s") * 2 + lax.axis_index("c")
  base = wid * BPW

  pltpu.sync_copy(u_hbm.at[wid], iu_s)
  pltpu.sync_copy(ip_hbm.at[wid], ii_s)
  pltpu.sync_copy(in_hbm.at[wid], ij_s)

  lane = lax.iota(jnp.int32, 16)

  def drain_row(_):
    pltpu.make_async_copy(
        utab_hbm.at[pl.ds(0, 1)], pu_v.at[pl.ds(0, 1)], sem).wait()
    pltpu.make_async_copy(
        itab_hbm.at[pl.ds(0, 1)], qi_v.at[pl.ds(0, 1)], sem).wait()
    pltpu.make_async_copy(
        itab_hbm.at[pl.ds(0, 1)], qj_v.at[pl.ds(0, 1)], sem).wait()

  def chunk_body(k, carry):
    ro = k * CH

    def fetch(gg, _):
      iu16 = iu_s[pl.ds(ro + gg * 16, 16)]
      ii16 = ii_s[pl.ds(ro + gg * 16, 16)]
      ij16 = ij_s[pl.ds(ro + gg * 16, 16)]
      for r in range(16):
        dst = pl.ds(gg * 16 + r, 1)
        pltpu.async_copy(utab_hbm.at[pl.ds(iu16[r], 1)], pu_v.at[dst], sem)
        pltpu.async_copy(itab_hbm.at[pl.ds(ii16[r], 1)], qi_v.at[dst], sem)
        pltpu.async_copy(itab_hbm.at[pl.ds(ij16[r], 1)], qj_v.at[dst], sem)
      return _

    lax.fori_loop(0, CH // 16, fetch, 0)
    lax.fori_loop(0, CH, lambda i, c: (drain_row(None), c)[1], 0)

    def body(g, c2):
      acc16 = jnp.zeros((16,), jnp.float32)
      for r in range(16):
        row = g * 16 + r
        s = jnp.zeros((16,), jnp.float32)
        for c in range(4):
          cs = pl.ds(c * 16, 16)
          s = s + pu_v[row, cs] * (qi_v[row, cs] - qj_v[row, cs])
        acc16 = jnp.where(lane == r, jnp.sum(s), acc16)
      x_v[pl.ds(ro + g * 16, 16)] = acc16
      return c2

    lax.fori_loop(0, CH // 16, body, 0)
    return carry

  lax.fori_loop(0, BPW // CH, chunk_body, 0)

  pltpu.sync_copy(x_v, x_hbm.at[pl.ds(base, BPW)])


@jax.jit
def _sc_stage(u, i_pos, i_neg, user_table, item_table):
  mesh = plsc.VectorSubcoreMesh(core_axis_name="c", subcore_axis_name="s")
  f = pl.kernel(
      _sc_dots,
      out_type=jax.ShapeDtypeStruct((BATCH,), jnp.float32),
      mesh=mesh,
      compiler_params=pltpu.CompilerParams(
          needs_layout_passes=False, use_tc_tiling_on_sc=True),
      scratch_types=[
          pltpu.VMEM((BPW,), jnp.int32),
          pltpu.VMEM((BPW,), jnp.int32),
          pltpu.VMEM((BPW,), jnp.int32),
          pltpu.VMEM((CH, DIM), jnp.float32),
          pltpu.VMEM((CH, DIM), jnp.float32),
          pltpu.VMEM((CH, DIM), jnp.float32),
          pltpu.VMEM((BPW,), jnp.float32),
          pltpu.SemaphoreType.DMA,
      ],
  )
  return f(u, i_pos, i_neg, user_table, item_table)


def _loss_body(x_ref, out_ref):
  x = x_ref[...]
  t = jnp.log(jax.nn.sigmoid(x) + 1e-08)
  out_ref[0, 0] = -jnp.sum(t) * (1.0 / BATCH)


@jax.jit
def _tc_loss(x):
  res = pl.pallas_call(
      _loss_body,
      out_shape=jax.ShapeDtypeStruct((1, 1), jnp.float32),
      out_specs=pl.BlockSpec(memory_space=pltpu.SMEM),
  )(x.reshape(128, 128))
  return res[0, 0]


def kernel(u, i_pos, i_neg, user_table, item_table):
  u = u.astype(jnp.int32).reshape(NW, BPW)
  i_pos = i_pos.astype(jnp.int32).reshape(NW, BPW)
  i_neg = i_neg.astype(jnp.int32).reshape(NW, BPW)
  x = _sc_stage(u, i_pos, i_neg, user_table, item_table)
  return _tc_loss(x)
